# Initial kernel scaffold; baseline (speedup 1.0000x reference)
import functools

import jax
import jax.numpy as jnp
from jax import lax
from jax.experimental import pallas as pl
from jax.experimental.pallas import tpu as pltpu
from jax.experimental.pallas import tpu_sc as plsc

N = 10000
E = 320000
D = 128
L = 7
NEG_SLOPE = 0.02
EPS = 1e-5

NW = 32
NPB = 313
NPAD = NW * NPB
CH = 128
NEG_INF = float("-inf")

_mesh = plsc.VectorSubcoreMesh(core_axis_name="c", subcore_axis_name="s")


@functools.partial(
    pl.kernel,
    out_type=jax.ShapeDtypeStruct((NPAD, D), jnp.float32),
    mesh=_mesh,
    scratch_types=[
        pltpu.VMEM((CH,), jnp.int32),
        pltpu.VMEM((CH,), jnp.int32),
        pltpu.VMEM((CH, D), jnp.float32),
        pltpu.VMEM((NPB, D), jnp.float32),
        pltpu.VMEM((NW,), jnp.int32),
        pltpu.VMEM((NW,), jnp.int32),
        pltpu.SemaphoreType.DMA,
    ],
)
def _sc_segmax(h_hbm, src_hbm, dst_hbm, starts_hbm, counts_hbm, agg_hbm,
               idx_v, dst_v, rows_v, agg_v, starts_v, counts_v, sem):
    wid = lax.axis_index("s") * 2 + lax.axis_index("c")

    pltpu.sync_copy(starts_hbm, starts_v)
    pltpu.sync_copy(counts_hbm, counts_v)
    base = starts_v[wid]
    cnt = counts_v[wid]

    neg = jnp.full((16,), NEG_INF, jnp.float32)

    @pl.loop(0, NPB)
    def _(r):
        for j in range(D // 16):
            agg_v[r, pl.ds(16 * j, 16)] = neg

    ab = base - lax.rem(base, 8)
    lead = base - ab
    total = lead + cnt
    nch = lax.div(total + CH - 1, CH)

    def chunk_body(k, carry):
        off = ab + k * CH
        pltpu.sync_copy(src_hbm.at[pl.ds(off, CH)], idx_v)
        pltpu.sync_copy(dst_hbm.at[pl.ds(off, CH)], dst_v)
        pltpu.async_copy(h_hbm.at[idx_v], rows_v, sem).wait()
        e0 = jnp.maximum(lead - k * CH, 0)
        e1 = jnp.minimum(total - k * CH, CH)

        def edge_body(e, c2):
            d = dst_v[e] - wid * NPB
            for j in range(D // 16):
                sl = pl.ds(16 * j, 16)
                agg_v[d, sl] = jnp.maximum(agg_v[d, sl], rows_v[e, sl])
            return c2

        lax.fori_loop(e0, e1, edge_body, 0)
        return carry

    lax.fori_loop(0, nch, chunk_body, 0)

    pltpu.sync_copy(agg_v, agg_hbm.at[pl.ds(wid * NPB, NPB)])


def _dot(a, b):
    return lax.dot_general(a, b, (((1,), (0,)), ((), ())),
                           precision=lax.Precision.HIGHEST,
                           preferred_element_type=jnp.float32)


def _combine_mid_body(agg_ref, h_ref, wl_ref, wr_ref, b_ref, gs_ref, gw_ref,
                      gb_ref, o_ref):
    agg = agg_ref[...]
    agg = jnp.where(agg == NEG_INF, 0.0, agg)
    out = _dot(agg, wl_ref[...]) + b_ref[...] + _dot(h_ref[...], wr_ref[...])
    rows = lax.broadcasted_iota(jnp.int32, (NPAD, 1), 0)
    m = rows < N
    mean = jnp.sum(jnp.where(m, out, 0.0), axis=0, keepdims=True) * (1.0 / N)
    o = out - mean * gs_ref[...]
    var = jnp.sum(jnp.where(m, o * o, 0.0), axis=0, keepdims=True) * (1.0 / N)
    o = o * lax.rsqrt(var + EPS) * gw_ref[...] + gb_ref[...]
    o_ref[...] = jnp.where(o >= 0, o, o * NEG_SLOPE)


def _combine_last_body(agg_ref, h_ref, wl_ref, wr_ref, b_ref, o_ref):
    agg = agg_ref[...]
    agg = jnp.where(agg == NEG_INF, 0.0, agg)
    out = _dot(agg, wl_ref[...]) + b_ref[...] + _dot(h_ref[...], wr_ref[...])
    o_ref[...] = jnp.tanh(out) * 0.5


_out_struct = jax.ShapeDtypeStruct((NPAD, D), jnp.float32)

_combine_mid = pl.pallas_call(_combine_mid_body, out_shape=_out_struct)
_combine_last = pl.pallas_call(_combine_last_body, out_shape=_out_struct)


def kernel(x, edge_index, W_l, b, W_r, gn_weight, gn_bias, gn_mean_scale):
    src = edge_index[0].astype(jnp.int32)
    dst = edge_index[1].astype(jnp.int32)

    order = jnp.argsort(dst)
    src_s = jnp.concatenate([src[order], jnp.zeros((CH,), jnp.int32)])
    dst_s0 = dst[order]
    dst_s = jnp.concatenate([dst_s0, jnp.zeros((CH,), jnp.int32)])
    bnd = jnp.arange(NW + 1, dtype=jnp.int32) * NPB
    offs = jnp.searchsorted(dst_s0, bnd, side="left").astype(jnp.int32)
    starts = offs[:-1]
    counts = offs[1:] - offs[:-1]

    h = jnp.zeros((NPAD, D), jnp.float32).at[:N].set(x)
    b2 = b.reshape(L, 1, D)
    gw2 = gn_weight.reshape(L - 1, 1, D)
    gb2 = gn_bias.reshape(L - 1, 1, D)
    gs2 = gn_mean_scale.reshape(L - 1, 1, D)

    for i in range(L):
        agg = _sc_segmax(h, src_s, dst_s, starts, counts)
        if i < L - 1:
            h = _combine_mid(agg, h, W_l[i], W_r[i], b2[i], gs2[i], gw2[i],
                             gb2[i])
        else:
            h = _combine_last(agg, h, W_l[i], W_r[i], b2[i])
    return h[:N]


# speedup vs baseline: 2.4364x; 2.4364x over previous
import dataclasses
import functools

import jax
import jax.numpy as jnp
from jax import lax
from jax.experimental import pallas as pl
from jax.experimental.pallas import tpu as pltpu
from jax.experimental.pallas import tpu_sc as plsc

N = 10000
E = 320000
D = 128
L = 7
NEG_SLOPE = 0.02
EPS = 1e-5

NW = 32
NPB = 320
NPAD = NW * NPB
CH = 128
NEG_INF = float("-inf")

_mesh = plsc.VectorSubcoreMesh(core_axis_name="c", subcore_axis_name="s")

_sc_params = pltpu.CompilerParams()
if "needs_layout_passes" in pltpu.CompilerParams.__dataclass_fields__:
    _sc_params = dataclasses.replace(_sc_params, needs_layout_passes=False)


@functools.partial(
    pl.kernel,
    out_type=jax.ShapeDtypeStruct((NPAD, D), jnp.float32),
    mesh=_mesh,
    compiler_params=_sc_params,
    scratch_types=[
        pltpu.VMEM((CH,), jnp.int32),
        pltpu.VMEM((CH,), jnp.int32),
        pltpu.VMEM((CH, D), jnp.float32),
        pltpu.VMEM((NPB + 1, D), jnp.float32),
        pltpu.VMEM((2 * NW,), jnp.int32),
        pltpu.SemaphoreType.DMA,
    ],
)
def _sc_segmax(h_hbm, src_hbm, dst_hbm, sc_hbm, agg_hbm,
               idx_v, dst_v, rows_v, agg_v, sc_v, sem):
    wid = lax.axis_index("s") * 2 + lax.axis_index("c")

    pltpu.sync_copy(sc_hbm, sc_v)
    widv = jnp.full((16,), wid, jnp.int32)
    base = plsc.load_gather(sc_v, [widv])[0]
    cnt = plsc.load_gather(sc_v, [widv + NW])[0]

    neg = jnp.full((16,), NEG_INF, jnp.float32)

    @pl.loop(0, NPB + 1)
    def _(r):
        for j in range(D // 16):
            agg_v[r, pl.ds(16 * j, 16)] = neg

    ab = base - lax.rem(base, 8)
    total = (base - ab) + cnt
    nch = lax.div(total + CH - 1, CH)
    lo = wid * NPB

    def chunk_body(k, carry):
        off = pl.multiple_of(ab + k * CH, 8)
        pltpu.sync_copy(src_hbm.at[pl.ds(off, CH)], idx_v)
        pltpu.sync_copy(dst_hbm.at[pl.ds(off, CH)], dst_v)
        pltpu.async_copy(h_hbm.at[idx_v], rows_v, sem).wait()

        @pl.loop(0, CH // 16)
        def _(g):
            e = g * 16
            dvec = dst_v[pl.ds(e, 16)] - lo
            valid = (dvec >= 0) & (dvec < NPB)
            dvec = jnp.where(valid, dvec, NPB)
            for j in range(16):
                d = dvec[j]
                for f in range(D // 16):
                    sl = pl.ds(16 * f, 16)
                    agg_v[d, sl] = jnp.maximum(agg_v[d, sl],
                                               rows_v[e + j, sl])
        return carry

    lax.fori_loop(0, nch, chunk_body, 0)

    pltpu.sync_copy(agg_v.at[pl.ds(0, NPB)], agg_hbm.at[pl.ds(lo, NPB)])


def _dot(a, b):
    return lax.dot_general(a, b, (((1,), (0,)), ((), ())),
                           precision=lax.Precision.DEFAULT,
                           preferred_element_type=jnp.float32)


def _combine_mid_body(agg_ref, h_ref, wl_ref, wr_ref, b_ref, gs_ref, gw_ref,
                      gb_ref, o_ref):
    agg = agg_ref[...]
    agg = jnp.where(agg == NEG_INF, 0.0, agg)
    out = _dot(agg, wl_ref[...]) + b_ref[...] + _dot(h_ref[...], wr_ref[...])
    rows = lax.broadcasted_iota(jnp.int32, (NPAD, 1), 0)
    m = rows < N
    mean = jnp.sum(jnp.where(m, out, 0.0), axis=0, keepdims=True) * (1.0 / N)
    o = out - mean * gs_ref[...]
    var = jnp.sum(jnp.where(m, o * o, 0.0), axis=0, keepdims=True) * (1.0 / N)
    o = o * lax.rsqrt(var + EPS) * gw_ref[...] + gb_ref[...]
    o_ref[...] = jnp.where(o >= 0, o, o * NEG_SLOPE)


def _combine_last_body(agg_ref, h_ref, wl_ref, wr_ref, b_ref, o_ref):
    agg = agg_ref[...]
    agg = jnp.where(agg == NEG_INF, 0.0, agg)
    out = _dot(agg, wl_ref[...]) + b_ref[...] + _dot(h_ref[...], wr_ref[...])
    o_ref[...] = jnp.tanh(out) * 0.5


_out_struct = jax.ShapeDtypeStruct((NPAD, D), jnp.float32)

_combine_mid = pl.pallas_call(_combine_mid_body, out_shape=_out_struct)
_combine_last = pl.pallas_call(_combine_last_body, out_shape=_out_struct)


def kernel(x, edge_index, W_l, b, W_r, gn_weight, gn_bias, gn_mean_scale):
    src = edge_index[0].astype(jnp.int32)
    dst = edge_index[1].astype(jnp.int32)

    order = jnp.argsort(dst)
    src_s = jnp.concatenate([src[order], jnp.zeros((CH,), jnp.int32)])
    dst_s0 = dst[order]
    dst_s = jnp.concatenate(
        [dst_s0, jnp.full((CH,), 2 * NPAD, jnp.int32)])
    bnd = jnp.arange(NW + 1, dtype=jnp.int32) * NPB
    offs = jnp.searchsorted(dst_s0, bnd, side="left").astype(jnp.int32)
    sc = jnp.concatenate([offs[:-1], offs[1:] - offs[:-1]])

    h = jnp.zeros((NPAD, D), jnp.float32).at[:N].set(x)
    b2 = b.reshape(L, 1, D)
    gw2 = gn_weight.reshape(L - 1, 1, D)
    gb2 = gn_bias.reshape(L - 1, 1, D)
    gs2 = gn_mean_scale.reshape(L - 1, 1, D)

    for i in range(L):
        agg = _sc_segmax(h, src_s, dst_s, sc)
        if i < L - 1:
            h = _combine_mid(agg, h, W_l[i], W_r[i], b2[i], gs2[i], gw2[i],
                             gb2[i])
        else:
            h = _combine_last(agg, h, W_l[i], W_r[i], b2[i])
    return h[:N]


# speedup vs baseline: 4.4858x; 1.8412x over previous
import dataclasses
import functools

import jax
import jax.numpy as jnp
from jax import lax
from jax.experimental import pallas as pl
from jax.experimental.pallas import tpu as pltpu
from jax.experimental.pallas import tpu_sc as plsc

N = 10000
E = 320000
D = 128
L = 7
NEG_SLOPE = 0.02
EPS = 1e-5

NW = 32
NPB = 320
NPAD = NW * NPB
CH = 128
NEG_INF = float("-inf")

_mesh = plsc.VectorSubcoreMesh(core_axis_name="c", subcore_axis_name="s")

_sc_params = pltpu.CompilerParams()
for _f, _v in (("needs_layout_passes", False),
               ("use_tc_tiling_on_sc", False)):
    if _f in pltpu.CompilerParams.__dataclass_fields__:
        _sc_params = dataclasses.replace(_sc_params, **{_f: _v})


@functools.partial(
    pl.kernel,
    out_type=jax.ShapeDtypeStruct((NPAD, D // 2), jnp.int32),
    mesh=_mesh,
    compiler_params=_sc_params,
    scratch_types=[
        pltpu.VMEM((CH,), jnp.int32),
        pltpu.VMEM((CH,), jnp.int32),
        pltpu.VMEM((CH,), jnp.int32),
        pltpu.VMEM((CH,), jnp.int32),
        pltpu.VMEM((CH, D // 2), jnp.int32),
        pltpu.VMEM((CH, D // 2), jnp.int32),
        pltpu.VMEM((NPB + 1, D // 2), jnp.int32),
        pltpu.VMEM((2 * NW,), jnp.int32),
        pltpu.SemaphoreType.DMA,
        pltpu.SemaphoreType.DMA,
        pltpu.SemaphoreType.DMA,
        pltpu.SemaphoreType.DMA,
    ],
)
def _sc_segmax(h_hbm, src_hbm, dst_hbm, sc_hbm, agg_hbm,
               idx0, idx1, dst0, dst1, rows0, rows1, agg_v, sc_v,
               semg0, semg1, semi0, semi1):
    wid = lax.axis_index("s") * 2 + lax.axis_index("c")

    pltpu.sync_copy(sc_hbm, sc_v)
    widv = jnp.full((16,), wid, jnp.int32)
    base = plsc.load_gather(sc_v, [widv])[0]
    cnt = plsc.load_gather(sc_v, [widv + NW])[0]

    neg = plsc.bitcast(jnp.full((32,), NEG_INF, jnp.bfloat16), jnp.int32)

    @pl.loop(0, NPB + 1)
    def _(r):
        for j in range(D // 32):
            agg_v[r, pl.ds(16 * j, 16)] = neg

    ab = base - lax.rem(base, 8)
    total = (base - ab) + cnt
    nch = lax.div(total + CH - 1, CH)
    nit = lax.div(nch + 1, 2)
    lo = wid * NPB

    def off_of(k):
        return pl.multiple_of(ab + k * CH, 8)

    def fetch_idx(k, idx_v, semi):
        pltpu.make_async_copy(src_hbm.at[pl.ds(off_of(k), CH)], idx_v,
                              semi).start()

    def wait_idx(idx_v, semi):
        pltpu.make_async_copy(src_hbm.at[pl.ds(0, CH)], idx_v, semi).wait()

    def start_gather(k, idx_v, rows_v, dst_v, semg):
        pltpu.make_async_copy(h_hbm.at[idx_v], rows_v, semg).start()
        pltpu.make_async_copy(dst_hbm.at[pl.ds(off_of(k), CH)], dst_v,
                              semg).start()

    def wait_gather(idx_v, rows_v, dst_v, semg):
        pltpu.make_async_copy(h_hbm.at[idx_v], rows_v, semg).wait()
        pltpu.make_async_copy(dst_hbm.at[pl.ds(0, CH)], dst_v, semg).wait()

    def compute(rows_v, dst_v):
        @pl.loop(0, CH // 16)
        def _(g):
            e = g * 16
            dvec = dst_v[pl.ds(e, 16)] - lo
            valid = (dvec >= 0) & (dvec < NPB)
            dvec = jnp.where(valid, dvec, NPB)
            for j in range(16):
                d = dvec[j]
                for f in range(D // 32):
                    sl = pl.ds(16 * f, 16)
                    a = plsc.bitcast(agg_v[d, sl], jnp.bfloat16)
                    r = plsc.bitcast(rows_v[e + j, sl], jnp.bfloat16)
                    agg_v[d, sl] = plsc.bitcast(jnp.maximum(a, r), jnp.int32)

    pltpu.sync_copy(src_hbm.at[pl.ds(off_of(0), CH)], idx0)
    pltpu.sync_copy(src_hbm.at[pl.ds(off_of(1), CH)], idx1)
    start_gather(0, idx0, rows0, dst0, semg0)
    start_gather(1, idx1, rows1, dst1, semg1)
    fetch_idx(2, idx0, semi0)
    fetch_idx(3, idx1, semi1)

    def iter_body(t, carry):
        a = 2 * t
        wait_gather(idx0, rows0, dst0, semg0)
        compute(rows0, dst0)
        wait_idx(idx0, semi0)
        start_gather(a + 2, idx0, rows0, dst0, semg0)
        fetch_idx(a + 4, idx0, semi0)

        wait_gather(idx1, rows1, dst1, semg1)
        compute(rows1, dst1)
        wait_idx(idx1, semi1)
        start_gather(a + 3, idx1, rows1, dst1, semg1)
        fetch_idx(a + 5, idx1, semi1)
        return carry

    lax.fori_loop(0, nit, iter_body, 0)

    wait_gather(idx0, rows0, dst0, semg0)
    wait_gather(idx1, rows1, dst1, semg1)
    wait_idx(idx0, semi0)
    wait_idx(idx1, semi1)

    pltpu.sync_copy(agg_v.at[pl.ds(0, NPB)], agg_hbm.at[pl.ds(lo, NPB)])


def _dot(a, b):
    return lax.dot_general(a, b, (((1,), (0,)), ((), ())),
                           precision=lax.Precision.DEFAULT,
                           preferred_element_type=jnp.float32)


def _combine_mid_body(agg_ref, h_ref, wl_ref, wr_ref, b_ref, gs_ref, gw_ref,
                      gb_ref, o_ref, obf_ref):
    agg = agg_ref[...].astype(jnp.float32)
    agg = jnp.where(agg == NEG_INF, 0.0, agg)
    out = _dot(agg, wl_ref[...]) + b_ref[...] + _dot(h_ref[...], wr_ref[...])
    rows = lax.broadcasted_iota(jnp.int32, (NPAD, 1), 0)
    m = rows < N
    mean = jnp.sum(jnp.where(m, out, 0.0), axis=0, keepdims=True) * (1.0 / N)
    o = out - mean * gs_ref[...]
    var = jnp.sum(jnp.where(m, o * o, 0.0), axis=0, keepdims=True) * (1.0 / N)
    o = o * lax.rsqrt(var + EPS) * gw_ref[...] + gb_ref[...]
    h_next = jnp.where(o >= 0, o, o * NEG_SLOPE)
    o_ref[...] = h_next
    obf_ref[...] = h_next.astype(jnp.bfloat16)


def _combine_last_body(agg_ref, h_ref, wl_ref, wr_ref, b_ref, o_ref):
    agg = agg_ref[...].astype(jnp.float32)
    agg = jnp.where(agg == NEG_INF, 0.0, agg)
    out = _dot(agg, wl_ref[...]) + b_ref[...] + _dot(h_ref[...], wr_ref[...])
    o_ref[...] = jnp.tanh(out) * 0.5


_out_f32 = jax.ShapeDtypeStruct((NPAD, D), jnp.float32)
_out_bf16 = jax.ShapeDtypeStruct((NPAD, D), jnp.bfloat16)

_combine_mid = pl.pallas_call(_combine_mid_body,
                              out_shape=(_out_f32, _out_bf16))
_combine_last = pl.pallas_call(_combine_last_body, out_shape=_out_f32)


def kernel(x, edge_index, W_l, b, W_r, gn_weight, gn_bias, gn_mean_scale):
    src = edge_index[0].astype(jnp.int32)
    dst = edge_index[1].astype(jnp.int32)

    order = jnp.argsort(dst)
    src_s = jnp.concatenate([src[order], jnp.zeros((8 * CH,), jnp.int32)])
    dst_s0 = dst[order]
    dst_s = jnp.concatenate(
        [dst_s0, jnp.full((8 * CH,), 2 * NPAD, jnp.int32)])
    bnd = jnp.arange(NW + 1, dtype=jnp.int32) * NPB
    offs = jnp.searchsorted(dst_s0, bnd, side="left").astype(jnp.int32)
    sc = jnp.concatenate([offs[:-1], offs[1:] - offs[:-1]])

    h = jnp.zeros((NPAD, D), jnp.float32).at[:N].set(x)
    hb = h.astype(jnp.bfloat16)
    b2 = b.reshape(L, 1, D)

    def pack(hbf):
        return lax.bitcast_convert_type(
            hbf.reshape(NPAD, D // 2, 2), jnp.int32)

    def unpack(ai32):
        return lax.bitcast_convert_type(ai32, jnp.bfloat16).reshape(NPAD, D)
    gw2 = gn_weight.reshape(L - 1, 1, D)
    gb2 = gn_bias.reshape(L - 1, 1, D)
    gs2 = gn_mean_scale.reshape(L - 1, 1, D)

    for i in range(L):
        agg = unpack(_sc_segmax(pack(hb), src_s, dst_s, sc))
        if i < L - 1:
            h, hb = _combine_mid(agg, h, W_l[i], W_r[i], b2[i], gs2[i],
                                 gw2[i], gb2[i])
        else:
            h = _combine_last(agg, h, W_l[i], W_r[i], b2[i])
    return h[:N]


# speedup vs baseline: 4.8648x; 1.0845x over previous
import dataclasses
import functools

import jax
import jax.numpy as jnp
from jax import lax
from jax.experimental import pallas as pl
from jax.experimental.pallas import tpu as pltpu
from jax.experimental.pallas import tpu_sc as plsc

N = 10000
E = 320000
D = 128
L = 7
NEG_SLOPE = 0.02
EPS = 1e-5

NW = 32
NPB = 320
NPAD = NW * NPB
CH = 128
NEG_INF = float("-inf")

_mesh = plsc.VectorSubcoreMesh(core_axis_name="c", subcore_axis_name="s")

_sc_params = pltpu.CompilerParams()
for _f, _v in (("needs_layout_passes", False),
               ("use_tc_tiling_on_sc", False)):
    if _f in pltpu.CompilerParams.__dataclass_fields__:
        _sc_params = dataclasses.replace(_sc_params, **{_f: _v})


@functools.partial(
    pl.kernel,
    out_type=jax.ShapeDtypeStruct((NPAD, D), jnp.bfloat16),
    mesh=_mesh,
    compiler_params=_sc_params,
    scratch_types=[
        pltpu.VMEM((CH,), jnp.int32),
        pltpu.VMEM((CH,), jnp.int32),
        pltpu.VMEM((CH,), jnp.int32),
        pltpu.VMEM((CH,), jnp.int32),
        pltpu.VMEM((CH,), jnp.int32),
        pltpu.VMEM((CH,), jnp.int32),
        pltpu.VMEM((CH, D // 2), jnp.int32),
        pltpu.VMEM((CH, D // 2), jnp.int32),
        pltpu.VMEM((NPB + 1, D), jnp.bfloat16),
        pltpu.VMEM((2 * NW,), jnp.int32),
        pltpu.SemaphoreType.DMA,
        pltpu.SemaphoreType.DMA,
        pltpu.SemaphoreType.DMA,
        pltpu.SemaphoreType.DMA,
    ],
)
def _sc_segmax(h_hbm, ps_hbm, sc_hbm, agg_hbm,
               pk0, pk1, idx0, idx1, dst0, dst1, rows0, rows1, agg_v, sc_v,
               semg0, semg1, semp0, semp1):
    wid = lax.axis_index("s") * 2 + lax.axis_index("c")

    pltpu.sync_copy(sc_hbm, sc_v)
    widv = jnp.full((16,), wid, jnp.int32)
    base = plsc.load_gather(sc_v, [widv])[0]
    cnt = plsc.load_gather(sc_v, [widv + NW])[0]

    neg = jnp.full((32,), NEG_INF, jnp.bfloat16)

    @pl.loop(0, NPB + 1)
    def _(r):
        for j in range(D // 32):
            agg_v[r, pl.ds(32 * j, 32)] = neg

    ab = base - lax.rem(base, 8)
    total = (base - ab) + cnt
    nch = lax.div(total + CH - 1, CH)
    nit = lax.div(nch + 1, 2)
    lo = wid * NPB

    def fetch_pk(k, pk_v, semp):
        off = pl.multiple_of(ab + k * CH, 8)
        pltpu.make_async_copy(ps_hbm.at[pl.ds(off, CH)], pk_v, semp).start()

    def wait_pk(pk_v, semp):
        pltpu.make_async_copy(ps_hbm.at[pl.ds(0, CH)], pk_v, semp).wait()

    def unpack(pk_v, idx_v, dst_v):
        @pl.loop(0, CH // 16)
        def _(g):
            sl = pl.ds(g * 16, 16)
            pv = pk_v[sl]
            idx_v[sl] = pv & ((1 << 14) - 1)
            dst_v[sl] = lax.shift_right_logical(pv, 14)

    def start_gather(idx_v, rows_v, semg):
        pltpu.make_async_copy(h_hbm.at[idx_v], rows_v, semg).start()

    def wait_gather(idx_v, rows_v, semg):
        pltpu.make_async_copy(h_hbm.at[idx_v], rows_v, semg).wait()

    def compute(rows_v, dst_v):
        @pl.loop(0, CH // 16)
        def _(g):
            e = g * 16
            dvec = dst_v[pl.ds(e, 16)] - lo
            valid = (dvec >= 0) & (dvec < NPB)
            dvec = jnp.where(valid, dvec, NPB)
            for j in range(16):
                d = dvec[j]
                for f in range(D // 32):
                    a = agg_v[d, pl.ds(32 * f, 32)]
                    r = plsc.bitcast(rows_v[e + j, pl.ds(16 * f, 16)],
                                     jnp.bfloat16)
                    agg_v[d, pl.ds(32 * f, 32)] = jnp.maximum(a, r)

    fetch_pk(0, pk0, semp0)
    fetch_pk(1, pk1, semp1)
    wait_pk(pk0, semp0)
    unpack(pk0, idx0, dst0)
    start_gather(idx0, rows0, semg0)
    wait_pk(pk1, semp1)
    unpack(pk1, idx1, dst1)
    start_gather(idx1, rows1, semg1)
    fetch_pk(2, pk0, semp0)
    fetch_pk(3, pk1, semp1)

    def iter_body(t, carry):
        a = 2 * t
        wait_gather(idx0, rows0, semg0)
        compute(rows0, dst0)
        wait_pk(pk0, semp0)
        unpack(pk0, idx0, dst0)
        start_gather(idx0, rows0, semg0)
        fetch_pk(a + 4, pk0, semp0)

        wait_gather(idx1, rows1, semg1)
        compute(rows1, dst1)
        wait_pk(pk1, semp1)
        unpack(pk1, idx1, dst1)
        start_gather(idx1, rows1, semg1)
        fetch_pk(a + 5, pk1, semp1)
        return carry

    lax.fori_loop(0, nit, iter_body, 0)

    wait_gather(idx0, rows0, semg0)
    wait_gather(idx1, rows1, semg1)
    wait_pk(pk0, semp0)
    wait_pk(pk1, semp1)

    pltpu.sync_copy(agg_v.at[pl.ds(0, NPB)], agg_hbm.at[pl.ds(lo, NPB)])


def _dot(a, b):
    return lax.dot_general(a, b, (((1,), (0,)), ((), ())),
                           precision=lax.Precision.DEFAULT,
                           preferred_element_type=jnp.float32)


def _combine_mid_body(agg_ref, h_ref, wl_ref, wr_ref, b_ref, gs_ref, gw_ref,
                      gb_ref, o_ref, obf_ref):
    agg = agg_ref[...].astype(jnp.float32)
    agg = jnp.where(agg == NEG_INF, 0.0, agg)
    out = _dot(agg, wl_ref[...]) + b_ref[...] + _dot(h_ref[...], wr_ref[...])
    rows = lax.broadcasted_iota(jnp.int32, (NPAD, 1), 0)
    m = rows < N
    mean = jnp.sum(jnp.where(m, out, 0.0), axis=0, keepdims=True) * (1.0 / N)
    o = out - mean * gs_ref[...]
    var = jnp.sum(jnp.where(m, o * o, 0.0), axis=0, keepdims=True) * (1.0 / N)
    o = o * lax.rsqrt(var + EPS) * gw_ref[...] + gb_ref[...]
    h_next = jnp.where(o >= 0, o, o * NEG_SLOPE)
    o_ref[...] = h_next
    obf_ref[...] = h_next.astype(jnp.bfloat16)


def _combine_last_body(agg_ref, h_ref, wl_ref, wr_ref, b_ref, o_ref):
    agg = agg_ref[...].astype(jnp.float32)
    agg = jnp.where(agg == NEG_INF, 0.0, agg)
    out = _dot(agg, wl_ref[...]) + b_ref[...] + _dot(h_ref[...], wr_ref[...])
    o_ref[...] = jnp.tanh(out) * 0.5


_out_f32 = jax.ShapeDtypeStruct((NPAD, D), jnp.float32)
_out_bf16 = jax.ShapeDtypeStruct((NPAD, D), jnp.bfloat16)

_combine_mid = pl.pallas_call(_combine_mid_body,
                              out_shape=(_out_f32, _out_bf16))
_combine_last = pl.pallas_call(_combine_last_body, out_shape=_out_f32)


def kernel(x, edge_index, W_l, b, W_r, gn_weight, gn_bias, gn_mean_scale):
    src = edge_index[0].astype(jnp.int32)
    dst = edge_index[1].astype(jnp.int32)

    packed = jnp.sort((dst << 14) | src)
    ps = jnp.concatenate(
        [packed, jnp.full((8 * CH,), (2 * NPAD) << 14, jnp.int32)])
    bnd = (jnp.arange(NW + 1, dtype=jnp.int32) * NPB) << 14
    offs = jnp.searchsorted(packed, bnd, side="left").astype(jnp.int32)
    sc = jnp.concatenate([offs[:-1], offs[1:] - offs[:-1]])

    h = jnp.zeros((NPAD, D), jnp.float32).at[:N].set(x)
    hb = h.astype(jnp.bfloat16)
    b2 = b.reshape(L, 1, D)

    def pack(hbf):
        return lax.bitcast_convert_type(
            hbf.reshape(NPAD, D // 2, 2), jnp.int32)

    gw2 = gn_weight.reshape(L - 1, 1, D)
    gb2 = gn_bias.reshape(L - 1, 1, D)
    gs2 = gn_mean_scale.reshape(L - 1, 1, D)

    for i in range(L):
        agg = _sc_segmax(pack(hb), ps, sc)
        if i < L - 1:
            h, hb = _combine_mid(agg, h, W_l[i], W_r[i], b2[i], gs2[i],
                                 gw2[i], gb2[i])
        else:
            h = _combine_last(agg, h, W_l[i], W_r[i], b2[i])
    return h[:N]


# speedup vs baseline: 6.1716x; 1.2686x over previous
import dataclasses
import functools

import jax
import jax.numpy as jnp
from jax import lax
from jax.experimental import pallas as pl
from jax.experimental.pallas import tpu as pltpu
from jax.experimental.pallas import tpu_sc as plsc

N = 10000
E = 320000
D = 128
L = 7
NEG_SLOPE = 0.02
EPS = 1e-5

NW = 32
NPB = 320
NPAD = NW * NPB
CH = 128
NEG_INF = float("-inf")

_mesh = plsc.VectorSubcoreMesh(core_axis_name="c", subcore_axis_name="s")

_sc_params = pltpu.CompilerParams()
for _f, _v in (("needs_layout_passes", False),
               ("use_tc_tiling_on_sc", False)):
    if _f in pltpu.CompilerParams.__dataclass_fields__:
        _sc_params = dataclasses.replace(_sc_params, **{_f: _v})


@functools.partial(
    pl.kernel,
    out_type=jax.ShapeDtypeStruct((NPAD, D), jnp.bfloat16),
    mesh=_mesh,
    compiler_params=_sc_params,
    scratch_types=[
        pltpu.VMEM((CH,), jnp.int32),
        pltpu.VMEM((CH,), jnp.int32),
        pltpu.VMEM((CH,), jnp.int32),
        pltpu.VMEM((CH,), jnp.int32),
        pltpu.VMEM((CH,), jnp.int32),
        pltpu.VMEM((CH,), jnp.int32),
        pltpu.VMEM((CH, D // 2), jnp.int32),
        pltpu.VMEM((CH, D // 2), jnp.int32),
        pltpu.VMEM((NPB + 1, D), jnp.bfloat16),
        pltpu.VMEM((2 * NW,), jnp.int32),
        pltpu.SemaphoreType.DMA,
        pltpu.SemaphoreType.DMA,
        pltpu.SemaphoreType.DMA,
        pltpu.SemaphoreType.DMA,
    ],
)
def _sc_segmax(h_hbm, ps_hbm, sc_hbm, agg_hbm,
               pk0, pk1, idx0, idx1, dst0, dst1, rows0, rows1, agg_v, sc_v,
               semg0, semg1, semp0, semp1):
    wid = lax.axis_index("s") * 2 + lax.axis_index("c")

    pltpu.sync_copy(sc_hbm, sc_v)
    widv = jnp.full((16,), wid, jnp.int32)
    base = plsc.load_gather(sc_v, [widv])[0]
    cnt = plsc.load_gather(sc_v, [widv + NW])[0]

    neg = jnp.full((32,), NEG_INF, jnp.bfloat16)

    @pl.loop(0, NPB + 1)
    def _(r):
        for j in range(D // 32):
            agg_v[r, pl.ds(32 * j, 32)] = neg

    ab = base - lax.rem(base, 8)
    total = (base - ab) + cnt
    nch = lax.div(total + CH - 1, CH)
    nit = lax.div(nch + 1, 2)
    lo = wid * NPB

    def fetch_pk(k, pk_v, semp):
        off = pl.multiple_of(ab + k * CH, 8)
        pltpu.make_async_copy(ps_hbm.at[pl.ds(off, CH)], pk_v, semp).start()

    def wait_pk(pk_v, semp):
        pltpu.make_async_copy(ps_hbm.at[pl.ds(0, CH)], pk_v, semp).wait()

    def unpack(pk_v, idx_v, dst_v):
        @pl.loop(0, CH // 16)
        def _(g):
            sl = pl.ds(g * 16, 16)
            pv = pk_v[sl]
            idx_v[sl] = pv & ((1 << 14) - 1)
            dst_v[sl] = lax.shift_right_logical(pv, 14)

    def start_gather(idx_v, rows_v, semg):
        pltpu.make_async_copy(h_hbm.at[idx_v], rows_v, semg).start()

    def wait_gather(idx_v, rows_v, semg):
        pltpu.make_async_copy(h_hbm.at[idx_v], rows_v, semg).wait()

    def compute(rows_v, dst_v):
        @pl.loop(0, CH // 16)
        def _(g):
            e = g * 16
            dvec = dst_v[pl.ds(e, 16)] - lo
            valid = (dvec >= 0) & (dvec < NPB)
            dvec = jnp.where(valid, dvec, NPB)
            d0 = dvec[0]

            def pure_group():
                for f in range(D // 32):
                    vals = [plsc.bitcast(rows_v[e + j, pl.ds(16 * f, 16)],
                                         jnp.bfloat16) for j in range(16)]
                    while len(vals) > 1:
                        vals = [jnp.maximum(vals[i], vals[i + 1])
                                for i in range(0, len(vals), 2)]
                    sl = pl.ds(32 * f, 32)
                    agg_v[d0, sl] = jnp.maximum(agg_v[d0, sl], vals[0])
                return 0

            def mixed_group():
                for j in range(16):
                    d = dvec[j]
                    for f in range(D // 32):
                        a = agg_v[d, pl.ds(32 * f, 32)]
                        r = plsc.bitcast(rows_v[e + j, pl.ds(16 * f, 16)],
                                         jnp.bfloat16)
                        agg_v[d, pl.ds(32 * f, 32)] = jnp.maximum(a, r)
                return 0

            lax.cond(d0 == dvec[15], pure_group, mixed_group)

    fetch_pk(0, pk0, semp0)
    fetch_pk(1, pk1, semp1)
    wait_pk(pk0, semp0)
    unpack(pk0, idx0, dst0)
    start_gather(idx0, rows0, semg0)
    wait_pk(pk1, semp1)
    unpack(pk1, idx1, dst1)
    start_gather(idx1, rows1, semg1)
    fetch_pk(2, pk0, semp0)
    fetch_pk(3, pk1, semp1)

    def iter_body(t, carry):
        a = 2 * t
        wait_gather(idx0, rows0, semg0)
        compute(rows0, dst0)
        wait_pk(pk0, semp0)
        unpack(pk0, idx0, dst0)
        start_gather(idx0, rows0, semg0)
        fetch_pk(a + 4, pk0, semp0)

        wait_gather(idx1, rows1, semg1)
        compute(rows1, dst1)
        wait_pk(pk1, semp1)
        unpack(pk1, idx1, dst1)
        start_gather(idx1, rows1, semg1)
        fetch_pk(a + 5, pk1, semp1)
        return carry

    lax.fori_loop(0, nit, iter_body, 0)

    wait_gather(idx0, rows0, semg0)
    wait_gather(idx1, rows1, semg1)
    wait_pk(pk0, semp0)
    wait_pk(pk1, semp1)

    pltpu.sync_copy(agg_v.at[pl.ds(0, NPB)], agg_hbm.at[pl.ds(lo, NPB)])


def _dot(a, b):
    return lax.dot_general(a, b, (((1,), (0,)), ((), ())),
                           precision=lax.Precision.DEFAULT,
                           preferred_element_type=jnp.float32)


def _combine_mid_body(agg_ref, h_ref, wl_ref, wr_ref, b_ref, gs_ref, gw_ref,
                      gb_ref, o_ref, obf_ref):
    agg = agg_ref[...].astype(jnp.float32)
    agg = jnp.where(agg == NEG_INF, 0.0, agg)
    out = _dot(agg, wl_ref[...]) + b_ref[...] + _dot(h_ref[...], wr_ref[...])
    rows = lax.broadcasted_iota(jnp.int32, (NPAD, 1), 0)
    m = rows < N
    mean = jnp.sum(jnp.where(m, out, 0.0), axis=0, keepdims=True) * (1.0 / N)
    o = out - mean * gs_ref[...]
    var = jnp.sum(jnp.where(m, o * o, 0.0), axis=0, keepdims=True) * (1.0 / N)
    o = o * lax.rsqrt(var + EPS) * gw_ref[...] + gb_ref[...]
    h_next = jnp.where(o >= 0, o, o * NEG_SLOPE)
    o_ref[...] = h_next
    obf_ref[...] = h_next.astype(jnp.bfloat16)


def _combine_last_body(agg_ref, h_ref, wl_ref, wr_ref, b_ref, o_ref):
    agg = agg_ref[...].astype(jnp.float32)
    agg = jnp.where(agg == NEG_INF, 0.0, agg)
    out = _dot(agg, wl_ref[...]) + b_ref[...] + _dot(h_ref[...], wr_ref[...])
    o_ref[...] = jnp.tanh(out) * 0.5


_out_f32 = jax.ShapeDtypeStruct((NPAD, D), jnp.float32)
_out_bf16 = jax.ShapeDtypeStruct((NPAD, D), jnp.bfloat16)

_combine_mid = pl.pallas_call(_combine_mid_body,
                              out_shape=(_out_f32, _out_bf16))
_combine_last = pl.pallas_call(_combine_last_body, out_shape=_out_f32)


def kernel(x, edge_index, W_l, b, W_r, gn_weight, gn_bias, gn_mean_scale):
    src = edge_index[0].astype(jnp.int32)
    dst = edge_index[1].astype(jnp.int32)

    packed = jnp.sort((dst << 14) | src)
    ps = jnp.concatenate(
        [packed, jnp.full((8 * CH,), (2 * NPAD) << 14, jnp.int32)])
    bnd = (jnp.arange(NW + 1, dtype=jnp.int32) * NPB) << 14
    offs = jnp.searchsorted(packed, bnd, side="left").astype(jnp.int32)
    sc = jnp.concatenate([offs[:-1], offs[1:] - offs[:-1]])

    h = jnp.zeros((NPAD, D), jnp.float32).at[:N].set(x)
    hb = h.astype(jnp.bfloat16)
    b2 = b.reshape(L, 1, D)

    def pack(hbf):
        return lax.bitcast_convert_type(
            hbf.reshape(NPAD, D // 2, 2), jnp.int32)

    gw2 = gn_weight.reshape(L - 1, 1, D)
    gb2 = gn_bias.reshape(L - 1, 1, D)
    gs2 = gn_mean_scale.reshape(L - 1, 1, D)

    for i in range(L):
        agg = _sc_segmax(pack(hb), ps, sc)
        if i < L - 1:
            h, hb = _combine_mid(agg, h, W_l[i], W_r[i], b2[i], gs2[i],
                                 gw2[i], gb2[i])
        else:
            h = _combine_last(agg, h, W_l[i], W_r[i], b2[i])
    return h[:N]


# speedup vs baseline: 6.7248x; 1.0896x over previous
import dataclasses
import functools

import jax
import jax.numpy as jnp
from jax import lax
from jax.experimental import pallas as pl
from jax.experimental.pallas import tpu as pltpu
from jax.experimental.pallas import tpu_sc as plsc

N = 10000
E = 320000
D = 128
L = 7
NEG_SLOPE = 0.02
EPS = 1e-5

NW = 32
NPB = 320
NPAD = NW * NPB
CH = 128
NEG_INF = float("-inf")

_mesh = plsc.VectorSubcoreMesh(core_axis_name="c", subcore_axis_name="s")

_sc_params = pltpu.CompilerParams()
for _f, _v in (("needs_layout_passes", False),
               ("use_tc_tiling_on_sc", False)):
    if _f in pltpu.CompilerParams.__dataclass_fields__:
        _sc_params = dataclasses.replace(_sc_params, **{_f: _v})


@functools.partial(
    pl.kernel,
    out_type=jax.ShapeDtypeStruct((NPAD, D), jnp.bfloat16),
    mesh=_mesh,
    compiler_params=_sc_params,
    scratch_types=[
        pltpu.VMEM((CH,), jnp.int32),
        pltpu.VMEM((CH,), jnp.int32),
        pltpu.VMEM((CH,), jnp.int32),
        pltpu.VMEM((CH,), jnp.int32),
        pltpu.VMEM((CH,), jnp.int32),
        pltpu.VMEM((CH,), jnp.int32),
        pltpu.VMEM((CH, D // 2), jnp.int32),
        pltpu.VMEM((CH, D // 2), jnp.int32),
        pltpu.VMEM((NPB + 1, D), jnp.bfloat16),
        pltpu.VMEM((2 * NW,), jnp.int32),
        pltpu.SemaphoreType.DMA,
        pltpu.SemaphoreType.DMA,
        pltpu.SemaphoreType.DMA,
        pltpu.SemaphoreType.DMA,
    ],
)
def _sc_segmax(h_hbm, ps_hbm, sc_hbm, agg_hbm,
               pk0, pk1, idx0, idx1, dst0, dst1, rows0, rows1, agg_v, sc_v,
               semg0, semg1, semp0, semp1):
    wid = lax.axis_index("s") * 2 + lax.axis_index("c")

    pltpu.sync_copy(sc_hbm, sc_v)
    widv = jnp.full((16,), wid, jnp.int32)
    base = plsc.load_gather(sc_v, [widv])[0]
    cnt = plsc.load_gather(sc_v, [widv + NW])[0]

    neg = jnp.full((32,), NEG_INF, jnp.bfloat16)

    @pl.loop(0, NPB + 1)
    def _(r):
        for j in range(D // 32):
            agg_v[r, pl.ds(32 * j, 32)] = neg

    ab = base - lax.rem(base, 8)
    total = (base - ab) + cnt
    nch = lax.div(total + CH - 1, CH)
    nit = lax.div(nch + 1, 2)
    lo = wid * NPB

    def fetch_pk(k, pk_v, semp):
        off = pl.multiple_of(ab + k * CH, 8)
        pltpu.make_async_copy(ps_hbm.at[pl.ds(off, CH)], pk_v, semp).start()

    def wait_pk(pk_v, semp):
        pltpu.make_async_copy(ps_hbm.at[pl.ds(0, CH)], pk_v, semp).wait()

    def unpack(pk_v, idx_v, dst_v):
        @pl.loop(0, CH // 16)
        def _(g):
            sl = pl.ds(g * 16, 16)
            pv = pk_v[sl]
            idx_v[sl] = pv & ((1 << 14) - 1)
            dst_v[sl] = lax.shift_right_logical(pv, 14)

    def start_gather(idx_v, rows_v, semg):
        pltpu.make_async_copy(h_hbm.at[idx_v], rows_v, semg).start()

    def wait_gather(idx_v, rows_v, semg):
        pltpu.make_async_copy(h_hbm.at[idx_v], rows_v, semg).wait()

    def compute(rows_v, dst_v):
        @pl.loop(0, CH // 16)
        def _(g):
            e = g * 16
            dvec = dst_v[pl.ds(e, 16)] - lo
            valid = (dvec >= 0) & (dvec < NPB)
            dvec = jnp.where(valid, dvec, NPB)
            d0 = dvec[0]

            def pure_group():
                for f in range(D // 32):
                    vals = [plsc.bitcast(rows_v[e + j, pl.ds(16 * f, 16)],
                                         jnp.bfloat16) for j in range(16)]
                    while len(vals) > 1:
                        vals = [jnp.maximum(vals[i], vals[i + 1])
                                for i in range(0, len(vals), 2)]
                    sl = pl.ds(32 * f, 32)
                    agg_v[d0, sl] = jnp.maximum(agg_v[d0, sl], vals[0])
                return 0

            def mixed_group():
                for q in range(4):
                    dq = dvec[4 * q]

                    def pure_quad(q=q, dq=dq):
                        for f in range(D // 32):
                            vals = [plsc.bitcast(
                                rows_v[e + 4 * q + j, pl.ds(16 * f, 16)],
                                jnp.bfloat16) for j in range(4)]
                            v = jnp.maximum(jnp.maximum(vals[0], vals[1]),
                                            jnp.maximum(vals[2], vals[3]))
                            sl = pl.ds(32 * f, 32)
                            agg_v[dq, sl] = jnp.maximum(agg_v[dq, sl], v)
                        return 0

                    def slow_quad(q=q):
                        for j in range(4 * q, 4 * q + 4):
                            d = dvec[j]
                            for f in range(D // 32):
                                a = agg_v[d, pl.ds(32 * f, 32)]
                                r = plsc.bitcast(
                                    rows_v[e + j, pl.ds(16 * f, 16)],
                                    jnp.bfloat16)
                                agg_v[d, pl.ds(32 * f, 32)] = jnp.maximum(a, r)
                        return 0

                    lax.cond(dq == dvec[4 * q + 3], pure_quad, slow_quad)
                return 0

            lax.cond(d0 == dvec[15], pure_group, mixed_group)

    fetch_pk(0, pk0, semp0)
    fetch_pk(1, pk1, semp1)
    wait_pk(pk0, semp0)
    unpack(pk0, idx0, dst0)
    start_gather(idx0, rows0, semg0)
    wait_pk(pk1, semp1)
    unpack(pk1, idx1, dst1)
    start_gather(idx1, rows1, semg1)
    fetch_pk(2, pk0, semp0)
    fetch_pk(3, pk1, semp1)

    def iter_body(t, carry):
        a = 2 * t
        wait_gather(idx0, rows0, semg0)
        compute(rows0, dst0)
        wait_pk(pk0, semp0)
        unpack(pk0, idx0, dst0)
        start_gather(idx0, rows0, semg0)
        fetch_pk(a + 4, pk0, semp0)

        wait_gather(idx1, rows1, semg1)
        compute(rows1, dst1)
        wait_pk(pk1, semp1)
        unpack(pk1, idx1, dst1)
        start_gather(idx1, rows1, semg1)
        fetch_pk(a + 5, pk1, semp1)
        return carry

    lax.fori_loop(0, nit, iter_body, 0)

    wait_gather(idx0, rows0, semg0)
    wait_gather(idx1, rows1, semg1)
    wait_pk(pk0, semp0)
    wait_pk(pk1, semp1)

    pltpu.sync_copy(agg_v.at[pl.ds(0, NPB)], agg_hbm.at[pl.ds(lo, NPB)])


def _dot(a, b):
    return lax.dot_general(a, b, (((1,), (0,)), ((), ())),
                           precision=lax.Precision.DEFAULT,
                           preferred_element_type=jnp.float32)


def _combine_mid_body(agg_ref, h_ref, wl_ref, wr_ref, b_ref, gs_ref, gw_ref,
                      gb_ref, o_ref, obf_ref):
    agg = agg_ref[...].astype(jnp.float32)
    agg = jnp.where(agg == NEG_INF, 0.0, agg)
    out = _dot(agg, wl_ref[...]) + b_ref[...] + _dot(h_ref[...], wr_ref[...])
    rows = lax.broadcasted_iota(jnp.int32, (NPAD, 1), 0)
    m = rows < N
    mean = jnp.sum(jnp.where(m, out, 0.0), axis=0, keepdims=True) * (1.0 / N)
    o = out - mean * gs_ref[...]
    var = jnp.sum(jnp.where(m, o * o, 0.0), axis=0, keepdims=True) * (1.0 / N)
    o = o * lax.rsqrt(var + EPS) * gw_ref[...] + gb_ref[...]
    h_next = jnp.where(o >= 0, o, o * NEG_SLOPE)
    o_ref[...] = h_next
    obf_ref[...] = h_next.astype(jnp.bfloat16)


def _combine_last_body(agg_ref, h_ref, wl_ref, wr_ref, b_ref, o_ref):
    agg = agg_ref[...].astype(jnp.float32)
    agg = jnp.where(agg == NEG_INF, 0.0, agg)
    out = _dot(agg, wl_ref[...]) + b_ref[...] + _dot(h_ref[...], wr_ref[...])
    o_ref[...] = jnp.tanh(out) * 0.5


_out_f32 = jax.ShapeDtypeStruct((NPAD, D), jnp.float32)
_out_bf16 = jax.ShapeDtypeStruct((NPAD, D), jnp.bfloat16)

_combine_mid = pl.pallas_call(_combine_mid_body,
                              out_shape=(_out_f32, _out_bf16))
_combine_last = pl.pallas_call(_combine_last_body, out_shape=_out_f32)


def kernel(x, edge_index, W_l, b, W_r, gn_weight, gn_bias, gn_mean_scale):
    src = edge_index[0].astype(jnp.int32)
    dst = edge_index[1].astype(jnp.int32)

    packed = jnp.sort((dst << 14) | src)
    ps = jnp.concatenate(
        [packed, jnp.full((8 * CH,), (2 * NPAD) << 14, jnp.int32)])
    bnd = (jnp.arange(NW + 1, dtype=jnp.int32) * NPB) << 14
    offs = jnp.searchsorted(packed, bnd, side="left").astype(jnp.int32)
    sc = jnp.concatenate([offs[:-1], offs[1:] - offs[:-1]])

    h = jnp.zeros((NPAD, D), jnp.float32).at[:N].set(x)
    hb = h.astype(jnp.bfloat16)
    b2 = b.reshape(L, 1, D)

    def pack(hbf):
        return lax.bitcast_convert_type(
            hbf.reshape(NPAD, D // 2, 2), jnp.int32)

    gw2 = gn_weight.reshape(L - 1, 1, D)
    gb2 = gn_bias.reshape(L - 1, 1, D)
    gs2 = gn_mean_scale.reshape(L - 1, 1, D)

    for i in range(L):
        agg = _sc_segmax(pack(hb), ps, sc)
        if i < L - 1:
            h, hb = _combine_mid(agg, h, W_l[i], W_r[i], b2[i], gs2[i],
                                 gw2[i], gb2[i])
        else:
            h = _combine_last(agg, h, W_l[i], W_r[i], b2[i])
    return h[:N]
